# baseline (speedup 1.0000x reference)
import dataclasses
import functools

import jax
import jax.numpy as jnp
from jax import lax
from jax.experimental import pallas as pl
from jax.experimental.pallas import tpu as pltpu
from jax.experimental.pallas import tpu_sc as plsc

N = 10000
NNZ = 160000
E = 5000
D = 256
DA = 384

NC = 2
NS = 16
NW = NC * NS

CHUNK = 128
NCHUNK = NNZ // CHUNK
EPAD = NW * 160
NPAD = NW * 320
CAND = 2048
CBUF = CAND + 272


def _matmul(X, W):
    MB = 400

    def body(x_ref, w_ref, o_ref):
        o_ref[:, 0:D] = jnp.dot(x_ref[...], w_ref[...],
                                preferred_element_type=jnp.float32,
                                precision=lax.Precision.HIGHEST)
        o_ref[:, D:DA] = jnp.ones((MB, DA - D), jnp.float32)

    return pl.pallas_call(
        body,
        grid=(N // MB,),
        in_specs=[pl.BlockSpec((MB, D), lambda i: (i, 0)),
                  pl.BlockSpec((D, D), lambda i: (0, 0))],
        out_specs=pl.BlockSpec((MB, DA), lambda i: (i, 0)),
        out_shape=jax.ShapeDtypeStruct((N, DA), jnp.float32),
    )(X, W)


def _make_segment_sum(rpw, width, total_rows):
    mesh = plsc.VectorSubcoreMesh(core_axis_name="c", subcore_axis_name="s")
    cp = pltpu.CompilerParams()
    if "needs_layout_passes" in pltpu.CompilerParams.__dataclass_fields__:
        cp = dataclasses.replace(cp, needs_layout_passes=False)
    nk = width // 16
    dump = rpw
    acc_rows = rpw + 8

    @functools.partial(
        pl.kernel,
        out_type=jax.ShapeDtypeStruct((total_rows, width), jnp.float32),
        mesh=mesh,
        compiler_params=cp,
        scratch_types=[
            pltpu.VMEM((CHUNK,), jnp.int32),
            pltpu.VMEM((CHUNK,), jnp.int32),
            pltpu.VMEM((CBUF,), jnp.int32),
            pltpu.VMEM((CBUF,), jnp.int32),
            pltpu.VMEM((CHUNK, width), jnp.float32),
            pltpu.VMEM((acc_rows, width), jnp.float32),
            pltpu.SemaphoreType.DMA,
        ],
    )
    def k(table_hbm, key_hbm, gid_hbm, zacc_hbm, out_hbm,
          kv, gv, cand_g, cand_r, rows, acc, sem):
        c = lax.axis_index("c")
        s = lax.axis_index("s")
        w = c * NS + s
        lo = w * rpw

        pltpu.sync_copy(zacc_hbm, acc)

        def accum_chunk(start):
            pltpu.async_copy(
                table_hbm.at[cand_g.at[pl.ds(start, CHUNK)]],
                rows, sem).wait()

            @pl.loop(0, CHUNK // 16)
            def _(g):
                le16 = cand_r[pl.ds(start + g * 16, 16)]
                for j2 in range(16):
                    le_b = le16.at[jnp.full((16,), j2, jnp.int32)].get(
                        mode="promise_in_bounds")
                    jb = g * 16 + j2
                    for kcol in range(nk):
                        col = lax.iota(jnp.int32, 16) + kcol * 16
                        plsc.addupdate_scatter(
                            acc, [le_b, col], rows[jb, pl.ds(kcol * 16, 16)])

        def scan_body(ch, off):
            pltpu.sync_copy(key_hbm.at[ch], kv)
            pltpu.sync_copy(gid_hbm.at[ch], gv)

            def group(g, off):
                k16 = kv[pl.ds(g * 16, 16)]
                g16 = gv[pl.ds(g * 16, 16)]
                l16 = k16 - lo
                m = (l16 >= 0) & (l16 < rpw)
                mi = m.astype(jnp.int32)
                csum = plsc.cumsum(mi)
                pos = jnp.where(m, off + csum - 1,
                                CBUF - 16 + lax.iota(jnp.int32, 16))
                plsc.store_scatter(cand_g, [pos], g16)
                plsc.store_scatter(cand_r, [pos], l16)
                return off + csum[15]

            off = lax.fori_loop(0, CHUNK // 16, group, off, unroll=False)

            @pl.when(off >= CAND)
            def _():
                @pl.loop(0, CAND // CHUNK)
                def _(f):
                    accum_chunk(f * CHUNK)
                @pl.loop(0, 256 // 16)
                def _(t):
                    cand_g[pl.ds(t * 16, 16)] = cand_g[pl.ds(CAND + t * 16, 16)]
                    cand_r[pl.ds(t * 16, 16)] = cand_r[pl.ds(CAND + t * 16, 16)]

            return jnp.where(off >= CAND, off - CAND, off)

        off = lax.fori_loop(0, NCHUNK, scan_body, jnp.int32(0), unroll=False)

        @pl.loop(0, 9)
        def _(t):
            cand_g[pl.ds(off + t * 16, 16)] = jnp.zeros((16,), jnp.int32)
            cand_r[pl.ds(off + t * 16, 16)] = jnp.full((16,), dump, jnp.int32)

        nf = (off + CHUNK - 1) // CHUNK

        def tail(f, _):
            accum_chunk(f * CHUNK)
            return 0

        lax.fori_loop(0, nf, tail, 0, unroll=False)

        pltpu.sync_copy(acc.at[pl.ds(0, rpw)], out_hbm.at[pl.ds(lo, rpw)])

    return k


def _combine(sums, degE):
    RB = 200

    def body(p_ref, de_ref, o_ref):
        s = p_ref[...]
        cnt = s[:, D:D + 1]
        o_ref[...] = s[:, 0:D] / jnp.maximum(cnt, 1.0) * de_ref[...]

    return pl.pallas_call(
        body,
        grid=(E // RB,),
        in_specs=[pl.BlockSpec((RB, DA), lambda i: (i, 0)),
                  pl.BlockSpec((RB, 1), lambda i: (i, 0))],
        out_specs=pl.BlockSpec((RB, D), lambda i: (i, 0)),
        out_shape=jax.ShapeDtypeStruct((E, D), jnp.float32),
    )(sums, degE)


def _finalize(xv_raw, degV):
    RB = 200
    nb = N // RB

    def body(a_ref, dv_ref, o_ref):
        xv = a_ref[...] * dv_ref[...]
        nrm = jnp.sqrt(jnp.sum(xv * xv, axis=1, keepdims=True))
        scale = jnp.where(nrm > 0, 1.0 / nrm, 0.0)
        o_ref[...] = xv * scale

    return pl.pallas_call(
        body,
        grid=(nb,),
        in_specs=[pl.BlockSpec((RB, D), lambda i: (i, 0)),
                  pl.BlockSpec((RB, 1), lambda i: (i, 0))],
        out_specs=pl.BlockSpec((RB, D), lambda i: (i, 0)),
        out_shape=jax.ShapeDtypeStruct((N, D), jnp.float32),
    )(xv_raw, degV)


_edge_sums = _make_segment_sum(EPAD // NW, DA, EPAD)
_vertex_sums = _make_segment_sum(NPAD // NW, D, NPAD)


def kernel(X, vertex, edges, W, degE, degV):
    Xpa = _matmul(X, W)
    v2d = vertex.reshape(NCHUNK, CHUNK)
    e2d = edges.reshape(NCHUNK, CHUNK)
    zE = jnp.zeros((EPAD // NW + 8, DA), jnp.float32)
    zV = jnp.zeros((NPAD // NW + 8, D), jnp.float32)
    sums = _edge_sums(Xpa, e2d, v2d, zE)
    Xe = _combine(sums, degE)
    xv_raw = _vertex_sums(Xe, v2d, e2d, zV)
    return _finalize(xv_raw[:N], degV)


# speedup vs baseline: 2.2651x; 2.2651x over previous
import dataclasses
import functools

import jax
import jax.numpy as jnp
from jax import lax
from jax.experimental import pallas as pl
from jax.experimental.pallas import tpu as pltpu
from jax.experimental.pallas import tpu_sc as plsc

N = 10000
NNZ = 160000
E = 5000
D = 256
DA = 384

NC = 2
NS = 16
NW = NC * NS

CHUNK = 128
NCHUNK = NNZ // CHUNK
EPAD = NW * 160
NPAD = NW * 320
CAND = 1024
CBUF = CAND + 272
BX = 25
NBLK = NCHUNK // BX


def _matmul(X, W):
    MB = 400

    def body(x_ref, w_ref, o_ref):
        o_ref[:, 0:D] = jnp.dot(x_ref[...], w_ref[...],
                                preferred_element_type=jnp.float32,
                                precision=lax.Precision.HIGHEST)
        o_ref[:, D:DA] = jnp.ones((MB, DA - D), jnp.float32)

    return pl.pallas_call(
        body,
        grid=(N // MB,),
        in_specs=[pl.BlockSpec((MB, D), lambda i: (i, 0)),
                  pl.BlockSpec((D, D), lambda i: (0, 0))],
        out_specs=pl.BlockSpec((MB, DA), lambda i: (i, 0)),
        out_shape=jax.ShapeDtypeStruct((N, DA), jnp.float32),
    )(X, W)


def _make_segment_sum(rpw, width, total_rows):
    mesh = plsc.VectorSubcoreMesh(core_axis_name="c", subcore_axis_name="s")
    cp = pltpu.CompilerParams()
    if "needs_layout_passes" in pltpu.CompilerParams.__dataclass_fields__:
        cp = dataclasses.replace(cp, needs_layout_passes=False)
    nk = width // 16
    dump = rpw
    acc_rows = rpw + 8

    @functools.partial(
        pl.kernel,
        out_type=jax.ShapeDtypeStruct((total_rows, width), jnp.float32),
        mesh=mesh,
        compiler_params=cp,
        scratch_types=[
            pltpu.VMEM((BX, CHUNK), jnp.int32),
            pltpu.VMEM((BX, CHUNK), jnp.int32),
            pltpu.VMEM((CBUF,), jnp.int32),
            pltpu.VMEM((CBUF,), jnp.int32),
            pltpu.VMEM((CHUNK, width), jnp.float32),
            pltpu.VMEM((acc_rows, width), jnp.float32),
            pltpu.SemaphoreType.DMA,
        ],
    )
    def k(table_hbm, key_hbm, gid_hbm, zacc_hbm, out_hbm,
          kv, gv, cand_g, cand_r, rows, acc, sem):
        c = lax.axis_index("c")
        s = lax.axis_index("s")
        w = c * NS + s
        lo = w * rpw

        pltpu.sync_copy(zacc_hbm, acc)

        def accum_chunk(start):
            pltpu.async_copy(
                table_hbm.at[cand_g.at[pl.ds(start, CHUNK)]],
                rows, sem).wait()

            @pl.loop(0, CHUNK // 16)
            def _(g):
                le16 = cand_r[pl.ds(start + g * 16, 16)]
                for j2 in range(16):
                    le_b = le16.at[jnp.full((16,), j2, jnp.int32)].get(
                        mode="promise_in_bounds")
                    jb = g * 16 + j2
                    for kcol in range(nk):
                        col = lax.iota(jnp.int32, 16) + kcol * 16
                        plsc.addupdate_scatter(
                            acc, [le_b, col], rows[jb, pl.ds(kcol * 16, 16)])

        def scan_blk(blk, off):
            pltpu.sync_copy(key_hbm.at[blk], kv)
            pltpu.sync_copy(gid_hbm.at[blk], gv)

            def chunk_body(r, off):
                def group(g, off):
                    k16 = kv[r, pl.ds(g * 16, 16)]
                    g16 = gv[r, pl.ds(g * 16, 16)]
                    l16 = k16 - lo
                    m = (l16 >= 0) & (l16 < rpw)
                    mi = m.astype(jnp.int32)
                    csum = plsc.cumsum(mi)
                    pos = jnp.where(m, off + csum - 1,
                                    CBUF - 16 + lax.iota(jnp.int32, 16))
                    plsc.store_scatter(cand_g, [pos], g16)
                    plsc.store_scatter(cand_r, [pos], l16)
                    return off + csum[15]

                off = lax.fori_loop(0, CHUNK // 16, group, off, unroll=False)

                @pl.when(off >= CAND)
                def _():
                    @pl.loop(0, CAND // CHUNK)
                    def _(f):
                        accum_chunk(f * CHUNK)
                    @pl.loop(0, 256 // 16)
                    def _(t):
                        cand_g[pl.ds(t * 16, 16)] = cand_g[pl.ds(CAND + t * 16, 16)]
                        cand_r[pl.ds(t * 16, 16)] = cand_r[pl.ds(CAND + t * 16, 16)]

                return jnp.where(off >= CAND, off - CAND, off)

            return lax.fori_loop(0, BX, chunk_body, off, unroll=False)

        off = lax.fori_loop(0, NBLK, scan_blk, jnp.int32(0), unroll=False)

        @pl.loop(0, 9)
        def _(t):
            cand_g[pl.ds(off + t * 16, 16)] = jnp.zeros((16,), jnp.int32)
            cand_r[pl.ds(off + t * 16, 16)] = jnp.full((16,), dump, jnp.int32)

        nf = (off + CHUNK - 1) // CHUNK

        def tail(f, _):
            accum_chunk(f * CHUNK)
            return 0

        lax.fori_loop(0, nf, tail, 0, unroll=False)

        pltpu.sync_copy(acc.at[pl.ds(0, rpw)], out_hbm.at[pl.ds(lo, rpw)])

    return k


def _combine(sums, degE):
    RB = 200

    def body(p_ref, de_ref, o_ref):
        s = p_ref[...]
        cnt = s[:, D:D + 1]
        o_ref[...] = s[:, 0:D] / jnp.maximum(cnt, 1.0) * de_ref[...]

    return pl.pallas_call(
        body,
        grid=(E // RB,),
        in_specs=[pl.BlockSpec((RB, DA), lambda i: (i, 0)),
                  pl.BlockSpec((RB, 1), lambda i: (i, 0))],
        out_specs=pl.BlockSpec((RB, D), lambda i: (i, 0)),
        out_shape=jax.ShapeDtypeStruct((E, D), jnp.float32),
    )(sums, degE)


def _finalize(xv_raw, degV):
    RB = 200
    nb = N // RB

    def body(a_ref, dv_ref, o_ref):
        xv = a_ref[...] * dv_ref[...]
        nrm = jnp.sqrt(jnp.sum(xv * xv, axis=1, keepdims=True))
        scale = jnp.where(nrm > 0, 1.0 / nrm, 0.0)
        o_ref[...] = xv * scale

    return pl.pallas_call(
        body,
        grid=(nb,),
        in_specs=[pl.BlockSpec((RB, D), lambda i: (i, 0)),
                  pl.BlockSpec((RB, 1), lambda i: (i, 0))],
        out_specs=pl.BlockSpec((RB, D), lambda i: (i, 0)),
        out_shape=jax.ShapeDtypeStruct((N, D), jnp.float32),
    )(xv_raw, degV)


_edge_sums = _make_segment_sum(EPAD // NW, DA, EPAD)
_vertex_sums = _make_segment_sum(NPAD // NW, D, NPAD)


def kernel(X, vertex, edges, W, degE, degV):
    Xpa = _matmul(X, W)
    v2d = vertex.reshape(NBLK, BX, CHUNK)
    e2d = edges.reshape(NBLK, BX, CHUNK)
    zE = jnp.zeros((EPAD // NW + 8, DA), jnp.float32)
    zV = jnp.zeros((NPAD // NW + 8, D), jnp.float32)
    sums = _edge_sums(Xpa, e2d, v2d, zE)
    Xe = _combine(sums, degE)
    xv_raw = _vertex_sums(Xe, v2d, e2d, zV)
    return _finalize(xv_raw[:N], degV)
